# Initial kernel scaffold; baseline (speedup 1.0000x reference)
import functools

import jax
import jax.numpy as jnp
from jax import lax
from jax.experimental import pallas as pl
from jax.experimental.pallas import tpu as pltpu
from jax.experimental.pallas import tpu_sc as plsc

NN = 10000
NI = 320000
NP = 10240
D1 = 128
D2 = 48
NC = 2
NS = 16
NW = NC * NS
TI = NI // NW
K = 125
NCH = TI // K
SLAB = NP // NS


def _make_sc_pass(D, with_deg):
  mesh = plsc.VectorSubcoreMesh(core_axis_name="c", subcore_axis_name="s")
  out_type = [jax.ShapeDtypeStruct((NC, NS, SLAB, D), jnp.float32)]
  scratch = [
      pltpu.VMEM((NCH, K), jnp.int32),
      pltpu.VMEM((NCH, K), jnp.int32),
      pltpu.VMEM((K, D), jnp.float32),
      pltpu.VMEM_SHARED((NP, D), jnp.float32),
      pltpu.SemaphoreType.DMA,
  ]
  if with_deg:
    out_type.append(jax.ShapeDtypeStruct((NC, NS, SLAB, 16), jnp.float32))
    scratch += [
        pltpu.VMEM((K, 16), jnp.float32),
        pltpu.VMEM_SHARED((NP, 16), jnp.float32),
    ]

  def body(src, gidx, sidx, zrow, *rest):
    if with_deg:
      (zdeg, ones, out_acc, out_deg, gv, sv, rows, acc, sem, onev, dacc) = rest
    else:
      (out_acc, gv, sv, rows, acc, sem) = rest
    cid = lax.axis_index("c")
    sid = lax.axis_index("s")
    wid = cid * NS + sid
    pltpu.sync_copy(zrow, acc.at[pl.ds(sid * SLAB, SLAB)])
    if with_deg:
      pltpu.sync_copy(zdeg, dacc.at[pl.ds(sid * SLAB, SLAB)])
      pltpu.sync_copy(ones, onev)
    pltpu.sync_copy(gidx.at[wid], gv)
    pltpu.sync_copy(sidx.at[wid], sv)
    plsc.subcore_barrier()

    def step(j, carry):
      pltpu.async_copy(src.at[gv.at[j]], rows, sem).wait()
      pltpu.sync_copy(rows, acc.at[sv.at[j]], add=True)
      if with_deg:
        pltpu.sync_copy(onev, dacc.at[sv.at[j]], add=True)
      return carry

    lax.fori_loop(0, NCH, step, 0)
    plsc.subcore_barrier()
    pltpu.sync_copy(acc.at[pl.ds(sid * SLAB, SLAB)], out_acc.at[cid, sid])
    if with_deg:
      pltpu.sync_copy(dacc.at[pl.ds(sid * SLAB, SLAB)], out_deg.at[cid, sid])

  return pl.kernel(body, mesh=mesh, out_type=out_type,
                   scratch_types=scratch)


_sc_pass_deg = _make_sc_pass(D1, True)
_sc_pass_48 = _make_sc_pass(D2, False)


def _mm_body(x_ref, w_ref, b_ref, o_ref):
  o_ref[...] = (jnp.dot(x_ref[...], w_ref[...],
                        preferred_element_type=jnp.float32) + b_ref[...])


def _mm(x, w, b, bs=1280):
  n, kdim = x.shape
  m = w.shape[1]
  return pl.pallas_call(
      _mm_body,
      grid=(n // bs,),
      in_specs=[pl.BlockSpec((bs, kdim), lambda i: (i, 0)),
                pl.BlockSpec((kdim, m), lambda i: (0, 0)),
                pl.BlockSpec((1, m), lambda i: (0, 0))],
      out_specs=pl.BlockSpec((bs, m), lambda i: (i, 0)),
      out_shape=jax.ShapeDtypeStruct((n, m), jnp.float32),
  )(x, w, b.reshape(1, -1))


def _norm_body(relu, p_ref, d_ref, o_ref):
  s = p_ref[0] + p_ref[1]
  deg = jnp.maximum(d_ref[0, :, 0:1] + d_ref[1, :, 0:1], 1.0)
  r = s / deg
  o_ref[...] = jnp.maximum(r, 0.0) if relu else r


def _norm(p, d, relu, bs=1280):
  _, n, dim = p.shape
  return pl.pallas_call(
      functools.partial(_norm_body, relu),
      grid=(n // bs,),
      in_specs=[pl.BlockSpec((2, bs, dim), lambda i: (0, i, 0)),
                pl.BlockSpec((2, bs, 16), lambda i: (0, i, 0))],
      out_specs=pl.BlockSpec((bs, dim), lambda i: (i, 0)),
      out_shape=jax.ShapeDtypeStruct((n, dim), jnp.float32),
  )(p, d)


def _norm_mm_body(p_ref, d_ref, w_ref, b_ref, o_ref):
  s = p_ref[0] + p_ref[1]
  deg = jnp.maximum(d_ref[0, :, 0:1] + d_ref[1, :, 0:1], 1.0)
  h = jnp.maximum(s / deg, 0.0)
  o_ref[...] = (jnp.dot(h, w_ref[...],
                        preferred_element_type=jnp.float32) + b_ref[...])


def _norm_mm(p, d, w, b, bs=1280):
  _, n, kdim = p.shape
  m = w.shape[1]
  return pl.pallas_call(
      _norm_mm_body,
      grid=(n // bs,),
      in_specs=[pl.BlockSpec((2, bs, kdim), lambda i: (0, i, 0)),
                pl.BlockSpec((2, bs, 16), lambda i: (0, i, 0)),
                pl.BlockSpec((kdim, m), lambda i: (0, 0)),
                pl.BlockSpec((1, m), lambda i: (0, 0))],
      out_specs=pl.BlockSpec((bs, m), lambda i: (i, 0)),
      out_shape=jax.ShapeDtypeStruct((n, m), jnp.float32),
  )(p, d, w, b.reshape(1, -1))


@jax.jit
def kernel(x, hyperedge_index, W1, b1, W2, b2):
  idx = hyperedge_index.astype(jnp.int32)
  node_idx = idx[0].reshape(NW, NCH, K)
  edge_idx = idx[1].reshape(NW, NCH, K)

  zrow1 = jnp.zeros((SLAB, D1), jnp.float32)
  zrow2 = jnp.zeros((SLAB, D2), jnp.float32)
  zdeg = jnp.zeros((SLAB, 16), jnp.float32)
  ones = jnp.ones((K, 16), jnp.float32)

  xp = jnp.concatenate([x, jnp.zeros((NP - NN, x.shape[1]), jnp.float32)])
  w2p = jnp.concatenate(
      [W2, jnp.zeros((W2.shape[0], D2 - W2.shape[1]), jnp.float32)], axis=1)
  b2p = jnp.concatenate([b2, jnp.zeros((D2 - b2.shape[0],), jnp.float32)])

  x1 = _mm(xp, W1, b1)
  p_e, d_e = _sc_pass_deg(x1, node_idx, edge_idx, zrow1, zdeg, ones)
  p_e = p_e.reshape(NC, NP, D1)
  d_e = d_e.reshape(NC, NP, 16)
  e1 = _norm(p_e, d_e, relu=True)
  p_v, d_v = _sc_pass_deg(e1, edge_idx, node_idx, zrow1, zdeg, ones)
  p_v = p_v.reshape(NC, NP, D1)
  d_v = d_v.reshape(NC, NP, 16)

  x2 = _norm_mm(p_v, d_v, w2p, b2p)
  q_e = _sc_pass_48(x2, node_idx, edge_idx, zrow2).reshape(NC, NP, D2)
  e2 = _norm(q_e, d_e, relu=True)
  r_v = _sc_pass_48(e2, edge_idx, node_idx, zrow2).reshape(NC, NP, D2)
  out = _norm(r_v, d_v, relu=False)
  return out[:NN, :40]


# speedup vs baseline: 7.2342x; 7.2342x over previous
import functools

import jax
import jax.numpy as jnp
from jax import lax
from jax.experimental import pallas as pl
from jax.experimental.pallas import tpu as pltpu
from jax.experimental.pallas import tpu_sc as plsc

NN = 10000
NI = 320000
NP = 10240
D1 = 128
D2 = 128
NC = 2
NS = 16
NW = NC * NS
TI = NI // NW
K = 125
NCH = TI // K
SLAB = NP // NS


def _make_sc_pass(D):
  mesh = plsc.VectorSubcoreMesh(core_axis_name="c", subcore_axis_name="s")

  def body(src, gidx, sidx, zrow, out_acc, gv, sv, rows, acc, sem):
    cid = lax.axis_index("c")
    sid = lax.axis_index("s")
    wid = cid * NS + sid
    pltpu.sync_copy(zrow, acc.at[pl.ds(sid * SLAB, SLAB)])
    pltpu.sync_copy(gidx.at[wid], gv)
    pltpu.sync_copy(sidx.at[wid], sv)
    plsc.subcore_barrier()

    def step(j, carry):
      pltpu.async_copy(src.at[gv.at[j]], rows, sem).wait()
      pltpu.sync_copy(rows, acc.at[sv.at[j]], add=True)
      return carry

    lax.fori_loop(0, NCH, step, 0)
    plsc.subcore_barrier()
    pltpu.sync_copy(acc.at[pl.ds(sid * SLAB, SLAB)], out_acc.at[cid, sid])

  return pl.kernel(
      body, mesh=mesh,
      out_type=[jax.ShapeDtypeStruct((NC, NS, SLAB, D), jnp.float32)],
      scratch_types=[
          pltpu.VMEM((NCH, K), jnp.int32),
          pltpu.VMEM((NCH, K), jnp.int32),
          pltpu.VMEM((K, D), jnp.float32),
          pltpu.VMEM_SHARED((NP, D), jnp.float32),
          pltpu.SemaphoreType.DMA,
      ])


def _sc_degrees_body(stacked, ones, zrow, out_d, iv, onev, dacc):
  cid = lax.axis_index("c")
  sid = lax.axis_index("s")
  pltpu.sync_copy(zrow, dacc.at[pl.ds(sid * SLAB, SLAB)])
  pltpu.sync_copy(ones, onev)
  pltpu.sync_copy(stacked.at[cid, sid], iv)
  plsc.subcore_barrier()

  def step(j, carry):
    pltpu.sync_copy(onev, dacc.at[iv.at[0, j]], add=True)
    pltpu.sync_copy(onev, dacc.at[iv.at[1, j]], add=True)
    return carry

  lax.fori_loop(0, NCH, step, 0)
  plsc.subcore_barrier()
  pltpu.sync_copy(dacc.at[pl.ds(sid * SLAB, SLAB)], out_d.at[cid, sid])


_sc_degrees = pl.kernel(
    _sc_degrees_body,
    mesh=plsc.VectorSubcoreMesh(core_axis_name="c", subcore_axis_name="s"),
    out_type=[jax.ShapeDtypeStruct((NC, NS, SLAB, D1), jnp.float32)],
    scratch_types=[
        pltpu.VMEM((2, NCH, K), jnp.int32),
        pltpu.VMEM((K, D1), jnp.float32),
        pltpu.VMEM_SHARED((NP, D1), jnp.float32),
    ])

_sc_pass_128 = _make_sc_pass(D1)


def _mm_body(x_ref, w_ref, b_ref, o_ref):
  o_ref[...] = (jnp.dot(x_ref[...], w_ref[...],
                        preferred_element_type=jnp.float32) + b_ref[...])


def _mm(x, w, b, bs=1280):
  n, kdim = x.shape
  m = w.shape[1]
  return pl.pallas_call(
      _mm_body,
      grid=(n // bs,),
      in_specs=[pl.BlockSpec((bs, kdim), lambda i: (i, 0)),
                pl.BlockSpec((kdim, m), lambda i: (0, 0)),
                pl.BlockSpec((1, m), lambda i: (0, 0))],
      out_specs=pl.BlockSpec((bs, m), lambda i: (i, 0)),
      out_shape=jax.ShapeDtypeStruct((n, m), jnp.float32),
  )(x, w, b.reshape(1, -1))


def _norm_body(relu, p_ref, d_ref, o_ref):
  s = p_ref[0] + p_ref[1]
  deg = jnp.maximum(d_ref[:, 0:1], 1.0)
  r = s / deg
  o_ref[...] = jnp.maximum(r, 0.0) if relu else r


def _norm(p, d, relu, bs=1280):
  _, n, dim = p.shape
  return pl.pallas_call(
      functools.partial(_norm_body, relu),
      grid=(n // bs,),
      in_specs=[pl.BlockSpec((2, bs, dim), lambda i: (0, i, 0)),
                pl.BlockSpec((bs, D1), lambda i: (i, 0))],
      out_specs=pl.BlockSpec((bs, dim), lambda i: (i, 0)),
      out_shape=jax.ShapeDtypeStruct((n, dim), jnp.float32),
  )(p, d)


def _norm_mm_body(p_ref, d_ref, w_ref, b_ref, o_ref):
  s = p_ref[0] + p_ref[1]
  deg = jnp.maximum(d_ref[:, 0:1], 1.0)
  h = jnp.maximum(s / deg, 0.0)
  o_ref[...] = (jnp.dot(h, w_ref[...],
                        preferred_element_type=jnp.float32) + b_ref[...])


def _norm_mm(p, d, w, b, bs=1280):
  _, n, kdim = p.shape
  m = w.shape[1]
  return pl.pallas_call(
      _norm_mm_body,
      grid=(n // bs,),
      in_specs=[pl.BlockSpec((2, bs, kdim), lambda i: (0, i, 0)),
                pl.BlockSpec((bs, D1), lambda i: (i, 0)),
                pl.BlockSpec((kdim, m), lambda i: (0, 0)),
                pl.BlockSpec((1, m), lambda i: (0, 0))],
      out_specs=pl.BlockSpec((bs, m), lambda i: (i, 0)),
      out_shape=jax.ShapeDtypeStruct((n, m), jnp.float32),
  )(p, d, w, b.reshape(1, -1))


@jax.jit
def kernel(x, hyperedge_index, W1, b1, W2, b2):
  idx = hyperedge_index.astype(jnp.int32)
  node_idx = idx[0].reshape(NW, NCH, K)
  edge_idx = idx[1].reshape(NW, NCH, K)

  zrow1 = jnp.zeros((SLAB, D1), jnp.float32)
  ones = jnp.ones((K, D1), jnp.float32)

  xp = jnp.concatenate([x, jnp.zeros((NP - NN, x.shape[1]), jnp.float32)])
  w2p = jnp.concatenate(
      [W2, jnp.zeros((W2.shape[0], D2 - W2.shape[1]), jnp.float32)], axis=1)
  b2p = jnp.concatenate([b2, jnp.zeros((D2 - b2.shape[0],), jnp.float32)])

  stacked = jnp.stack([idx[0].reshape(NS, 2, NCH, K),
                       idx[1].reshape(NS, 2, NCH, K)])
  (d,) = _sc_degrees(stacked, ones, zrow1)
  d_v = d[0].reshape(NP, D1)
  d_e = d[1].reshape(NP, D1)

  x1 = _mm(xp, W1, b1)
  (p_e,) = _sc_pass_128(x1, node_idx, edge_idx, zrow1)
  e1 = _norm(p_e.reshape(NC, NP, D1), d_e, relu=True)
  (p_v,) = _sc_pass_128(e1, edge_idx, node_idx, zrow1)
  p_v = p_v.reshape(NC, NP, D1)

  x2 = _norm_mm(p_v, d_v, w2p, b2p)
  (q_e,) = _sc_pass_128(x2, node_idx, edge_idx, zrow1)
  e2 = _norm(q_e.reshape(NC, NP, D2), d_e, relu=True)
  (r_v,) = _sc_pass_128(e2, edge_idx, node_idx, zrow1)
  r_v = r_v.reshape(NC, NP, D2)
  out = _norm(r_v, d_v, relu=False)
  return out[:NN, :40]


# speedup vs baseline: 7.6439x; 1.0566x over previous
import functools

import jax
import jax.numpy as jnp
from jax import lax
from jax.experimental import pallas as pl
from jax.experimental.pallas import tpu as pltpu
from jax.experimental.pallas import tpu_sc as plsc

NN = 10000
NI = 320000
NP = 10240
D1 = 128
D2 = 128
NC = 2
NS = 16
NW = NC * NS
TI = NI // NW
K = 80
NCH = TI // K
NPH = 5
PH = NCH // NPH
SLAB = NP // NS


def _make_sc_pass(D):
  mesh = plsc.VectorSubcoreMesh(core_axis_name="c", subcore_axis_name="s")

  def body(src, gidx, sidx, zrow, out_acc, gv, sv, rows0, rows1, acc,
           sem0, sem1):
    cid = lax.axis_index("c")
    sid = lax.axis_index("s")
    wid = cid * NS + sid
    pltpu.sync_copy(zrow, acc.at[pl.ds(sid * SLAB, SLAB)])
    plsc.subcore_barrier()

    def run_phase(p, n):
      pltpu.sync_copy(gidx.at[wid, p], gv)
      pltpu.sync_copy(sidx.at[wid, p], sv)
      pltpu.async_copy(src.at[gv.at[0]], rows0, sem0)

      def pair(t, carry):
        j = 2 * t
        pltpu.make_async_copy(src.at[gv.at[j]], rows0, sem0).wait()
        pltpu.async_copy(src.at[gv.at[j + 1]], rows1, sem1)
        pltpu.sync_copy(rows0, acc.at[sv.at[j]], add=True)
        pltpu.make_async_copy(src.at[gv.at[j + 1]], rows1, sem1).wait()
        nxt = jnp.minimum(j + 2, n - 1)
        pltpu.async_copy(src.at[gv.at[nxt]], rows0, sem0)
        pltpu.sync_copy(rows1, acc.at[sv.at[j + 1]], add=True)
        return carry

      lax.fori_loop(0, (n - 1) // 2, pair, 0)
      if n % 2 == 1:
        pltpu.make_async_copy(src.at[gv.at[n - 1]], rows0, sem0).wait()
        pltpu.sync_copy(rows0, acc.at[sv.at[n - 1]], add=True)
      else:
        pltpu.make_async_copy(src.at[gv.at[n - 2]], rows0, sem0).wait()
        pltpu.async_copy(src.at[gv.at[n - 1]], rows1, sem1)
        pltpu.sync_copy(rows0, acc.at[sv.at[n - 2]], add=True)
        pltpu.make_async_copy(src.at[gv.at[n - 1]], rows1, sem1).wait()
        pltpu.sync_copy(rows1, acc.at[sv.at[n - 1]], add=True)

    for p in range(NPH):
      run_phase(p, PH)
    plsc.subcore_barrier()
    pltpu.sync_copy(acc.at[pl.ds(sid * SLAB, SLAB)], out_acc.at[cid, sid])

  return pl.kernel(
      body, mesh=mesh,
      out_type=[jax.ShapeDtypeStruct((NC, NS, SLAB, D), jnp.float32)],
      scratch_types=[
          pltpu.VMEM((PH, K), jnp.int32),
          pltpu.VMEM((PH, K), jnp.int32),
          pltpu.VMEM((K, D), jnp.float32),
          pltpu.VMEM((K, D), jnp.float32),
          pltpu.VMEM_SHARED((NP, D), jnp.float32),
          pltpu.SemaphoreType.DMA,
          pltpu.SemaphoreType.DMA,
      ])


def _sc_degrees_body(stacked, ones, zrow, out_d, iv, onev, dacc):
  cid = lax.axis_index("c")
  sid = lax.axis_index("s")
  pltpu.sync_copy(zrow, dacc.at[pl.ds(sid * SLAB, SLAB)])
  pltpu.sync_copy(ones, onev)
  pltpu.sync_copy(stacked.at[cid, sid], iv)
  plsc.subcore_barrier()

  def step(j, carry):
    pltpu.sync_copy(onev, dacc.at[iv.at[0, j]], add=True)
    pltpu.sync_copy(onev, dacc.at[iv.at[1, j]], add=True)
    return carry

  lax.fori_loop(0, NCH, step, 0)
  plsc.subcore_barrier()
  pltpu.sync_copy(dacc.at[pl.ds(sid * SLAB, SLAB)], out_d.at[cid, sid])


_sc_degrees = pl.kernel(
    _sc_degrees_body,
    mesh=plsc.VectorSubcoreMesh(core_axis_name="c", subcore_axis_name="s"),
    out_type=[jax.ShapeDtypeStruct((NC, NS, SLAB, D1), jnp.float32)],
    scratch_types=[
        pltpu.VMEM((2, NCH, K), jnp.int32),
        pltpu.VMEM((K, D1), jnp.float32),
        pltpu.VMEM_SHARED((NP, D1), jnp.float32),
    ])

_sc_pass_128 = _make_sc_pass(D1)


def _mm_body(x_ref, w_ref, b_ref, o_ref):
  o_ref[...] = (jnp.dot(x_ref[...], w_ref[...],
                        preferred_element_type=jnp.float32) + b_ref[...])


def _mm(x, w, b, bs=1280):
  n, kdim = x.shape
  m = w.shape[1]
  return pl.pallas_call(
      _mm_body,
      grid=(n // bs,),
      in_specs=[pl.BlockSpec((bs, kdim), lambda i: (i, 0)),
                pl.BlockSpec((kdim, m), lambda i: (0, 0)),
                pl.BlockSpec((1, m), lambda i: (0, 0))],
      out_specs=pl.BlockSpec((bs, m), lambda i: (i, 0)),
      out_shape=jax.ShapeDtypeStruct((n, m), jnp.float32),
  )(x, w, b.reshape(1, -1))


def _norm_body(relu, p_ref, d_ref, o_ref):
  s = p_ref[0] + p_ref[1]
  deg = jnp.maximum(d_ref[:, 0:1], 1.0)
  r = s / deg
  o_ref[...] = jnp.maximum(r, 0.0) if relu else r


def _norm(p, d, relu, bs=1280):
  _, n, dim = p.shape
  return pl.pallas_call(
      functools.partial(_norm_body, relu),
      grid=(n // bs,),
      in_specs=[pl.BlockSpec((2, bs, dim), lambda i: (0, i, 0)),
                pl.BlockSpec((bs, D1), lambda i: (i, 0))],
      out_specs=pl.BlockSpec((bs, dim), lambda i: (i, 0)),
      out_shape=jax.ShapeDtypeStruct((n, dim), jnp.float32),
  )(p, d)


def _norm_mm_body(p_ref, d_ref, w_ref, b_ref, o_ref):
  s = p_ref[0] + p_ref[1]
  deg = jnp.maximum(d_ref[:, 0:1], 1.0)
  h = jnp.maximum(s / deg, 0.0)
  o_ref[...] = (jnp.dot(h, w_ref[...],
                        preferred_element_type=jnp.float32) + b_ref[...])


def _norm_mm(p, d, w, b, bs=1280):
  _, n, kdim = p.shape
  m = w.shape[1]
  return pl.pallas_call(
      _norm_mm_body,
      grid=(n // bs,),
      in_specs=[pl.BlockSpec((2, bs, kdim), lambda i: (0, i, 0)),
                pl.BlockSpec((bs, D1), lambda i: (i, 0)),
                pl.BlockSpec((kdim, m), lambda i: (0, 0)),
                pl.BlockSpec((1, m), lambda i: (0, 0))],
      out_specs=pl.BlockSpec((bs, m), lambda i: (i, 0)),
      out_shape=jax.ShapeDtypeStruct((n, m), jnp.float32),
  )(p, d, w, b.reshape(1, -1))


@jax.jit
def kernel(x, hyperedge_index, W1, b1, W2, b2):
  idx = hyperedge_index.astype(jnp.int32)
  node_idx = idx[0].reshape(NW, NPH, PH, K)
  edge_idx = idx[1].reshape(NW, NPH, PH, K)

  zrow1 = jnp.zeros((SLAB, D1), jnp.float32)
  ones = jnp.ones((K, D1), jnp.float32)

  xp = jnp.concatenate([x, jnp.zeros((NP - NN, x.shape[1]), jnp.float32)])
  w2p = jnp.concatenate(
      [W2, jnp.zeros((W2.shape[0], D2 - W2.shape[1]), jnp.float32)], axis=1)
  b2p = jnp.concatenate([b2, jnp.zeros((D2 - b2.shape[0],), jnp.float32)])

  stacked = jnp.stack([idx[0].reshape(NS, 2, NCH, K),
                       idx[1].reshape(NS, 2, NCH, K)])
  (d,) = _sc_degrees(stacked, ones, zrow1)
  d_v = d[0].reshape(NP, D1)
  d_e = d[1].reshape(NP, D1)

  x1 = _mm(xp, W1, b1)
  (p_e,) = _sc_pass_128(x1, node_idx, edge_idx, zrow1)
  e1 = _norm(p_e.reshape(NC, NP, D1), d_e, relu=True)
  (p_v,) = _sc_pass_128(e1, edge_idx, node_idx, zrow1)
  p_v = p_v.reshape(NC, NP, D1)

  x2 = _norm_mm(p_v, d_v, w2p, b2p)
  (q_e,) = _sc_pass_128(x2, node_idx, edge_idx, zrow1)
  e2 = _norm(q_e.reshape(NC, NP, D2), d_e, relu=True)
  (r_v,) = _sc_pass_128(e2, edge_idx, node_idx, zrow1)
  r_v = r_v.reshape(NC, NP, D2)
  out = _norm(r_v, d_v, relu=False)
  return out[:NN, :40]


# speedup vs baseline: 8.6311x; 1.1291x over previous
import functools

import jax
import jax.numpy as jnp
from jax import lax
from jax.experimental import pallas as pl
from jax.experimental.pallas import tpu as pltpu
from jax.experimental.pallas import tpu_sc as plsc

NN = 10000
NI = 320000
NP = 10240
D1 = 128
D2 = 128
NC = 2
NS = 16
NW = NC * NS
TI = NI // NW
K = 125
NCH = TI // K
NPH = 5
PH = NCH // NPH
SLAB = NP // NS


def _make_sc_pass(D):
  mesh = plsc.VectorSubcoreMesh(core_axis_name="c", subcore_axis_name="s")

  def body(src, gidx, sidx, zrow, out_acc, gv, sv, rows0, rows1, acc,
           sem0, sem1):
    cid = lax.axis_index("c")
    sid = lax.axis_index("s")
    wid = cid * NS + sid
    pltpu.sync_copy(zrow, acc.at[pl.ds(sid * SLAB, SLAB)])
    plsc.subcore_barrier()

    def run_phase(p, n):
      pltpu.sync_copy(gidx.at[wid, p], gv)
      pltpu.sync_copy(sidx.at[wid, p], sv)
      pltpu.async_copy(src.at[gv.at[0]], rows0, sem0)

      def pair(t, carry):
        j = 2 * t
        pltpu.make_async_copy(src.at[gv.at[j]], rows0, sem0).wait()
        pltpu.async_copy(src.at[gv.at[j + 1]], rows1, sem1)
        pltpu.sync_copy(rows0, acc.at[sv.at[j]], add=True)
        pltpu.make_async_copy(src.at[gv.at[j + 1]], rows1, sem1).wait()
        nxt = jnp.minimum(j + 2, n - 1)
        pltpu.async_copy(src.at[gv.at[nxt]], rows0, sem0)
        pltpu.sync_copy(rows1, acc.at[sv.at[j + 1]], add=True)
        return carry

      lax.fori_loop(0, (n - 1) // 2, pair, 0)
      if n % 2 == 1:
        pltpu.make_async_copy(src.at[gv.at[n - 1]], rows0, sem0).wait()
        pltpu.sync_copy(rows0, acc.at[sv.at[n - 1]], add=True)
      else:
        pltpu.make_async_copy(src.at[gv.at[n - 2]], rows0, sem0).wait()
        pltpu.async_copy(src.at[gv.at[n - 1]], rows1, sem1)
        pltpu.sync_copy(rows0, acc.at[sv.at[n - 2]], add=True)
        pltpu.make_async_copy(src.at[gv.at[n - 1]], rows1, sem1).wait()
        pltpu.sync_copy(rows1, acc.at[sv.at[n - 1]], add=True)

    for p in range(NPH):
      run_phase(p, PH)
    plsc.subcore_barrier()
    pltpu.sync_copy(acc.at[pl.ds(sid * SLAB, SLAB)], out_acc.at[cid, sid])

  return pl.kernel(
      body, mesh=mesh,
      out_type=[jax.ShapeDtypeStruct((NC, NS, SLAB, D), jnp.float32)],
      scratch_types=[
          pltpu.VMEM((PH, K), jnp.int32),
          pltpu.VMEM((PH, K), jnp.int32),
          pltpu.VMEM((K, D), jnp.float32),
          pltpu.VMEM((K, D), jnp.float32),
          pltpu.VMEM_SHARED((NP, D), jnp.float32),
          pltpu.SemaphoreType.DMA,
          pltpu.SemaphoreType.DMA,
      ])


def _sc_degrees_body(stacked, ones, zrow, out_d, iv, onev, dacc):
  cid = lax.axis_index("c")
  sid = lax.axis_index("s")
  pltpu.sync_copy(zrow, dacc.at[pl.ds(sid * SLAB, SLAB)])
  pltpu.sync_copy(ones, onev)
  pltpu.sync_copy(stacked.at[cid, sid], iv)
  plsc.subcore_barrier()

  def step(j, carry):
    pltpu.sync_copy(onev, dacc.at[iv.at[0, j]], add=True)
    pltpu.sync_copy(onev, dacc.at[iv.at[1, j]], add=True)
    return carry

  lax.fori_loop(0, NCH, step, 0)
  plsc.subcore_barrier()
  pltpu.sync_copy(dacc.at[pl.ds(sid * SLAB, SLAB)], out_d.at[cid, sid])


_sc_degrees = pl.kernel(
    _sc_degrees_body,
    mesh=plsc.VectorSubcoreMesh(core_axis_name="c", subcore_axis_name="s"),
    out_type=[jax.ShapeDtypeStruct((NC, NS, SLAB, D1), jnp.float32)],
    scratch_types=[
        pltpu.VMEM((2, NCH, K), jnp.int32),
        pltpu.VMEM((K, D1), jnp.float32),
        pltpu.VMEM_SHARED((NP, D1), jnp.float32),
    ])

_sc_pass_128 = _make_sc_pass(D1)


def _mm_body(x_ref, w_ref, b_ref, o_ref):
  o_ref[...] = (jnp.dot(x_ref[...], w_ref[...],
                        preferred_element_type=jnp.float32) + b_ref[...])


def _mm(x, w, b, bs=1280):
  n, kdim = x.shape
  m = w.shape[1]
  return pl.pallas_call(
      _mm_body,
      grid=(n // bs,),
      in_specs=[pl.BlockSpec((bs, kdim), lambda i: (i, 0)),
                pl.BlockSpec((kdim, m), lambda i: (0, 0)),
                pl.BlockSpec((1, m), lambda i: (0, 0))],
      out_specs=pl.BlockSpec((bs, m), lambda i: (i, 0)),
      out_shape=jax.ShapeDtypeStruct((n, m), jnp.float32),
  )(x, w, b.reshape(1, -1))


def _norm_body(relu, p_ref, d_ref, o_ref):
  s = p_ref[0] + p_ref[1]
  deg = jnp.maximum(d_ref[:, 0:1], 1.0)
  r = s / deg
  o_ref[...] = jnp.maximum(r, 0.0) if relu else r


def _norm(p, d, relu, bs=1280):
  _, n, dim = p.shape
  return pl.pallas_call(
      functools.partial(_norm_body, relu),
      grid=(n // bs,),
      in_specs=[pl.BlockSpec((2, bs, dim), lambda i: (0, i, 0)),
                pl.BlockSpec((bs, D1), lambda i: (i, 0))],
      out_specs=pl.BlockSpec((bs, dim), lambda i: (i, 0)),
      out_shape=jax.ShapeDtypeStruct((n, dim), jnp.float32),
  )(p, d)


def _norm_mm_body(p_ref, d_ref, w_ref, b_ref, o_ref):
  s = p_ref[0] + p_ref[1]
  deg = jnp.maximum(d_ref[:, 0:1], 1.0)
  h = jnp.maximum(s / deg, 0.0)
  o_ref[...] = (jnp.dot(h, w_ref[...],
                        preferred_element_type=jnp.float32) + b_ref[...])


def _norm_mm(p, d, w, b, bs=1280):
  _, n, kdim = p.shape
  m = w.shape[1]
  return pl.pallas_call(
      _norm_mm_body,
      grid=(n // bs,),
      in_specs=[pl.BlockSpec((2, bs, kdim), lambda i: (0, i, 0)),
                pl.BlockSpec((bs, D1), lambda i: (i, 0)),
                pl.BlockSpec((kdim, m), lambda i: (0, 0)),
                pl.BlockSpec((1, m), lambda i: (0, 0))],
      out_specs=pl.BlockSpec((bs, m), lambda i: (i, 0)),
      out_shape=jax.ShapeDtypeStruct((n, m), jnp.float32),
  )(p, d, w, b.reshape(1, -1))


@jax.jit
def kernel(x, hyperedge_index, W1, b1, W2, b2):
  idx = hyperedge_index.astype(jnp.int32)
  node_idx = idx[0].reshape(NW, NPH, PH, K)
  edge_idx = idx[1].reshape(NW, NPH, PH, K)

  zrow1 = jnp.zeros((SLAB, D1), jnp.float32)
  ones = jnp.ones((K, D1), jnp.float32)

  xp = jnp.concatenate([x, jnp.zeros((NP - NN, x.shape[1]), jnp.float32)])
  w2p = jnp.concatenate(
      [W2, jnp.zeros((W2.shape[0], D2 - W2.shape[1]), jnp.float32)], axis=1)
  b2p = jnp.concatenate([b2, jnp.zeros((D2 - b2.shape[0],), jnp.float32)])

  stacked = jnp.stack([idx[0].reshape(NS, 2, NCH, K),
                       idx[1].reshape(NS, 2, NCH, K)])
  (d,) = _sc_degrees(stacked, ones, zrow1)
  d_v = d[0].reshape(NP, D1)
  d_e = d[1].reshape(NP, D1)

  x1 = _mm(xp, W1, b1)
  (p_e,) = _sc_pass_128(x1, node_idx, edge_idx, zrow1)
  e1 = _norm(p_e.reshape(NC, NP, D1), d_e, relu=True)
  (p_v,) = _sc_pass_128(e1, edge_idx, node_idx, zrow1)
  p_v = p_v.reshape(NC, NP, D1)

  x2 = _norm_mm(p_v, d_v, w2p, b2p)
  (q_e,) = _sc_pass_128(x2, node_idx, edge_idx, zrow1)
  e2 = _norm(q_e.reshape(NC, NP, D2), d_e, relu=True)
  (r_v,) = _sc_pass_128(e2, edge_idx, node_idx, zrow1)
  r_v = r_v.reshape(NC, NP, D2)
  out = _norm(r_v, d_v, relu=False)
  return out[:NN, :40]


# speedup vs baseline: 8.6409x; 1.0011x over previous
import functools

import jax
import jax.numpy as jnp
from jax import lax
from jax.experimental import pallas as pl
from jax.experimental.pallas import tpu as pltpu
from jax.experimental.pallas import tpu_sc as plsc

NN = 10000
NI = 320000
NP = 10240
D1 = 128
D2 = 128
NC = 2
NS = 16
NW = NC * NS
TI = NI // NW
K = 125
NCH = TI // K
NPH = 5
PH = NCH // NPH
SLAB = NP // NS


def _make_sc_pass(D):
  mesh = plsc.VectorSubcoreMesh(core_axis_name="c", subcore_axis_name="s")

  def body(src, gidx, sidx, zrow, out_acc, gv, sv, rows0, rows1, acc,
           sem0, sem1):
    cid = lax.axis_index("c")
    sid = lax.axis_index("s")
    wid = cid * NS + sid
    pltpu.sync_copy(zrow, acc.at[pl.ds(sid * SLAB, SLAB)])
    plsc.subcore_barrier()

    def run_phase(p, n):
      pltpu.sync_copy(gidx.at[wid, p], gv)
      pltpu.sync_copy(sidx.at[wid, p], sv)
      pltpu.async_copy(src.at[gv.at[0]], rows0, sem0)

      def pair(t, carry):
        j = 2 * t
        pltpu.make_async_copy(src.at[gv.at[j]], rows0, sem0).wait()
        pltpu.async_copy(src.at[gv.at[j + 1]], rows1, sem1)
        pltpu.sync_copy(rows0, acc.at[sv.at[j]], add=True)
        pltpu.make_async_copy(src.at[gv.at[j + 1]], rows1, sem1).wait()
        nxt = jnp.minimum(j + 2, n - 1)
        pltpu.async_copy(src.at[gv.at[nxt]], rows0, sem0)
        pltpu.sync_copy(rows1, acc.at[sv.at[j + 1]], add=True)
        return carry

      lax.fori_loop(0, (n - 1) // 2, pair, 0)
      if n % 2 == 1:
        pltpu.make_async_copy(src.at[gv.at[n - 1]], rows0, sem0).wait()
        pltpu.sync_copy(rows0, acc.at[sv.at[n - 1]], add=True)
      else:
        pltpu.make_async_copy(src.at[gv.at[n - 2]], rows0, sem0).wait()
        pltpu.async_copy(src.at[gv.at[n - 1]], rows1, sem1)
        pltpu.sync_copy(rows0, acc.at[sv.at[n - 2]], add=True)
        pltpu.make_async_copy(src.at[gv.at[n - 1]], rows1, sem1).wait()
        pltpu.sync_copy(rows1, acc.at[sv.at[n - 1]], add=True)

    for p in range(NPH):
      run_phase(p, PH)
    plsc.subcore_barrier()
    pltpu.sync_copy(acc.at[pl.ds(sid * SLAB, SLAB)], out_acc.at[cid, sid])

  return pl.kernel(
      body, mesh=mesh,
      out_type=[jax.ShapeDtypeStruct((NC, NS, SLAB, D), jnp.float32)],
      scratch_types=[
          pltpu.VMEM((PH, K), jnp.int32),
          pltpu.VMEM((PH, K), jnp.int32),
          pltpu.VMEM((K, D), jnp.float32),
          pltpu.VMEM((K, D), jnp.float32),
          pltpu.VMEM_SHARED((NP, D), jnp.float32),
          pltpu.SemaphoreType.DMA,
          pltpu.SemaphoreType.DMA,
      ])


def _sc_degrees_body(stacked, ones, zrow, out_d, iv, onev, dacc):
  cid = lax.axis_index("c")
  sid = lax.axis_index("s")
  pltpu.sync_copy(zrow, dacc.at[pl.ds(sid * SLAB, SLAB)])
  pltpu.sync_copy(ones, onev)
  pltpu.sync_copy(stacked.at[cid, sid], iv)
  plsc.subcore_barrier()

  def step(j, carry):
    pltpu.sync_copy(onev, dacc.at[iv.at[0, j]], add=True)
    pltpu.sync_copy(onev, dacc.at[iv.at[1, j]], add=True)
    return carry

  lax.fori_loop(0, NCH, step, 0)
  plsc.subcore_barrier()
  pltpu.sync_copy(dacc.at[pl.ds(sid * SLAB, SLAB)], out_d.at[cid, sid])


_sc_degrees = pl.kernel(
    _sc_degrees_body,
    mesh=plsc.VectorSubcoreMesh(core_axis_name="c", subcore_axis_name="s"),
    out_type=[jax.ShapeDtypeStruct((NC, NS, SLAB, D1), jnp.float32)],
    scratch_types=[
        pltpu.VMEM((2, NCH, K), jnp.int32),
        pltpu.VMEM((K, D1), jnp.float32),
        pltpu.VMEM_SHARED((NP, D1), jnp.float32),
    ])


_sc_pass_128 = _make_sc_pass(D1)


def _mm_body(x_ref, w_ref, b_ref, o_ref):
  o_ref[...] = (jnp.dot(x_ref[...], w_ref[...],
                        preferred_element_type=jnp.float32) + b_ref[...])


def _mm(x, w, b, bs=1280):
  n, kdim = x.shape
  m = w.shape[1]
  return pl.pallas_call(
      _mm_body,
      grid=(n // bs,),
      in_specs=[pl.BlockSpec((bs, kdim), lambda i: (i, 0)),
                pl.BlockSpec((kdim, m), lambda i: (0, 0)),
                pl.BlockSpec((1, m), lambda i: (0, 0))],
      out_specs=pl.BlockSpec((bs, m), lambda i: (i, 0)),
      out_shape=jax.ShapeDtypeStruct((n, m), jnp.float32),
  )(x, w, b.reshape(1, -1))


def _norm_body(relu, p_ref, d_ref, o_ref):
  s = p_ref[0] + p_ref[1]
  deg = jnp.maximum(d_ref[...], 1.0)
  r = s / deg
  o_ref[...] = jnp.maximum(r, 0.0) if relu else r


def _norm(p, d, relu, bs=1280):
  _, n, dim = p.shape
  return pl.pallas_call(
      functools.partial(_norm_body, relu),
      grid=(n // bs,),
      in_specs=[pl.BlockSpec((2, bs, dim), lambda i: (0, i, 0)),
                pl.BlockSpec((bs, 1), lambda i: (i, 0))],
      out_specs=pl.BlockSpec((bs, dim), lambda i: (i, 0)),
      out_shape=jax.ShapeDtypeStruct((n, dim), jnp.float32),
  )(p, d)


def _norm_mm_body(p_ref, d_ref, w_ref, b_ref, o_ref):
  s = p_ref[0] + p_ref[1]
  deg = jnp.maximum(d_ref[...], 1.0)
  h = jnp.maximum(s / deg, 0.0)
  o_ref[...] = (jnp.dot(h, w_ref[...],
                        preferred_element_type=jnp.float32) + b_ref[...])


def _norm_mm(p, d, w, b, bs=1280):
  _, n, kdim = p.shape
  m = w.shape[1]
  return pl.pallas_call(
      _norm_mm_body,
      grid=(n // bs,),
      in_specs=[pl.BlockSpec((2, bs, kdim), lambda i: (0, i, 0)),
                pl.BlockSpec((bs, 1), lambda i: (i, 0)),
                pl.BlockSpec((kdim, m), lambda i: (0, 0)),
                pl.BlockSpec((1, m), lambda i: (0, 0))],
      out_specs=pl.BlockSpec((bs, m), lambda i: (i, 0)),
      out_shape=jax.ShapeDtypeStruct((n, m), jnp.float32),
  )(p, d, w, b.reshape(1, -1))


@jax.jit
def kernel(x, hyperedge_index, W1, b1, W2, b2):
  idx = hyperedge_index.astype(jnp.int32)
  node_idx = idx[0].reshape(NW, NPH, PH, K)
  edge_idx = idx[1].reshape(NW, NPH, PH, K)

  zrow1 = jnp.zeros((SLAB, D1), jnp.float32)

  xp = jnp.concatenate([x, jnp.zeros((NP - NN, x.shape[1]), jnp.float32)])
  w2p = jnp.concatenate(
      [W2, jnp.zeros((W2.shape[0], D2 - W2.shape[1]), jnp.float32)], axis=1)
  b2p = jnp.concatenate([b2, jnp.zeros((D2 - b2.shape[0],), jnp.float32)])

  stacked = jnp.stack([idx[0].reshape(NS, 2, NCH, K),
                       idx[1].reshape(NS, 2, NCH, K)])
  ones = jnp.ones((K, D1), jnp.float32)
  (d,) = _sc_degrees(stacked, ones, zrow1)
  d_v = d[0].reshape(NP, D1)[:, 0:1]
  d_e = d[1].reshape(NP, D1)[:, 0:1]

  x1 = _mm(xp, W1, b1)
  (p_e,) = _sc_pass_128(x1, node_idx, edge_idx, zrow1)
  e1 = _norm(p_e.reshape(NC, NP, D1), d_e, relu=True)
  (p_v,) = _sc_pass_128(e1, edge_idx, node_idx, zrow1)
  p_v = p_v.reshape(NC, NP, D1)

  x2 = _norm_mm(p_v, d_v, w2p, b2p)
  (q_e,) = _sc_pass_128(x2, node_idx, edge_idx, zrow1)
  e2 = _norm(q_e.reshape(NC, NP, D2), d_e, relu=True)
  (r_v,) = _sc_pass_128(e2, edge_idx, node_idx, zrow1)
  r_v = r_v.reshape(NC, NP, D2)
  out = _norm(r_v, d_v, relu=False)
  return out[:NN, :40]
